# baseline (speedup 1.0000x reference)
import dataclasses
import functools

import jax
import jax.numpy as jnp
from jax import lax
from jax.experimental import pallas as pl
from jax.experimental.pallas import tpu as pltpu
from jax.experimental.pallas import tpu_sc as plsc

N = 10000
E = 160000
F = 256
H = 128
C = 16

NC = 2
NS = 16
LANES = 16

NPAD = 10240
_vmesh = plsc.VectorSubcoreMesh(core_axis_name="c", subcore_axis_name="s")


_DEG_EPT = E // (NC * NS)
_DEG_NCH = _DEG_EPT // LANES

_deg_cp = pltpu.CompilerParams()
if "needs_layout_passes" in pltpu.CompilerParams.__dataclass_fields__:
    _deg_cp = dataclasses.replace(_deg_cp, needs_layout_passes=False)


@jax.jit
def _sc_degree(dst):
    @functools.partial(
        pl.kernel,
        mesh=_vmesh,
        compiler_params=_deg_cp,
        out_type=jax.ShapeDtypeStruct((NC * NPAD,), jnp.float32),
        scratch_types=[
            pltpu.VMEM_SHARED((NS, NPAD), jnp.float32),
            pltpu.VMEM((NPAD,), jnp.float32),
            pltpu.VMEM((_DEG_EPT + 16,), jnp.int32),
            pltpu.VMEM((NS, NPAD // NS), jnp.float32),
            pltpu.VMEM((NPAD // NS,), jnp.float32),
            pltpu.SemaphoreType.DMA,
        ],
    )
    def k(dst_hbm, out_hbm, stage, hist, dstv, chunk, red, sem):
        c = lax.axis_index("c")
        s = lax.axis_index("s")
        wid = c * NS + s
        RPT = NPAD // NS

        cp = pltpu.async_copy(
            dst_hbm.at[pl.ds(wid * _DEG_EPT, _DEG_EPT)],
            dstv.at[pl.ds(0, _DEG_EPT)], sem)

        zero16 = jnp.zeros((LANES,), jnp.float32)

        @pl.loop(0, NPAD // LANES)
        def _(j):
            hist[pl.ds(j * LANES, LANES)] = zero16

        cp.wait()

        ones = jnp.full((LANES,), 1.0, jnp.float32)

        @pl.loop(0, _DEG_NCH)
        def _(j):
            idx = dstv[pl.ds(j * LANES, LANES)]
            plsc.addupdate_scatter(hist, [idx], ones)

        rem_idx = dstv[pl.ds(_DEG_NCH * LANES, LANES)]
        mask = lax.iota(jnp.int32, LANES) < (_DEG_EPT - _DEG_NCH * LANES)
        plsc.addupdate_scatter(hist, [rem_idx], ones, mask=mask)

        pltpu.sync_copy(hist, stage.at[s])
        plsc.subcore_barrier()
        pltpu.sync_copy(stage.at[:, pl.ds(s * RPT, RPT)], chunk)

        @pl.loop(0, RPT // LANES)
        def _(v):
            acc16 = chunk[0, pl.ds(v * LANES, LANES)]
            for r in range(1, NS):
                acc16 = acc16 + chunk[r, pl.ds(v * LANES, LANES)]
            red[pl.ds(v * LANES, LANES)] = acc16

        pltpu.sync_copy(red, out_hbm.at[pl.ds(c * NPAD + s * RPT, RPT)])

    return k(dst)


_P_EPT = E // NS
_P_B = 80
_P_NB = _P_EPT // _P_B
_P_RPT = NPAD // NS


@jax.jit
def _sc_propagate(xs_lo, xs_hi, src, dst):
    @functools.partial(
        pl.kernel,
        mesh=_vmesh,
        out_type=[
            jax.ShapeDtypeStruct((NPAD, H), jnp.float32),
            jax.ShapeDtypeStruct((NPAD, H), jnp.float32),
        ],
        scratch_types=[
            pltpu.VMEM_SHARED((NPAD, H), jnp.float32),
            pltpu.VMEM((_P_EPT,), jnp.int32),
            pltpu.VMEM((_P_EPT,), jnp.int32),
        ] + [pltpu.VMEM((_P_B,), jnp.int32) for _ in range(4)] + [
            pltpu.VMEM((_P_B, H), jnp.float32) for _ in range(2)
        ] + [pltpu.SemaphoreType.DMA for _ in range(6)],
    )
    def k(xlo_hbm, xhi_hbm, src_hbm, dst_hbm, olo_hbm, ohi_hbm,
          acc, srcv, dstv,
          sb0, db0, sb1, db1, rw0, rw1,
          semi, semz, sg0, sg1, ss0, ss1):
        c = lax.axis_index("c")
        s = lax.axis_index("s")
        ebase = s * _P_EPT
        rbase = s * _P_RPT
        srcb = [sb0, sb1]
        dstb = [db0, db1]
        rows = [rw0, rw1]
        semg = [sg0, sg1]
        sems = [ss0, ss1]

        def run(tab_hbm, out_hbm):
            cp_s = pltpu.async_copy(src_hbm.at[pl.ds(ebase, _P_EPT)], srcv, semi)
            cp_d = pltpu.async_copy(dst_hbm.at[pl.ds(ebase, _P_EPT)], dstv, semi)

            @pl.loop(0, _P_B)
            def _(r):
                for j in range(H // LANES):
                    rows[0][r, pl.ds(j * LANES, LANES)] = jnp.zeros(
                        (LANES,), jnp.float32
                    )

            zcps = [
                pltpu.async_copy(
                    rows[0], acc.at[pl.ds(rbase + i * _P_B, _P_B)], semz)
                for i in range(_P_RPT // _P_B)
            ]
            for cp in zcps:
                cp.wait()

            cp_s.wait()
            cp_d.wait()
            plsc.subcore_barrier()

            def fill(b, p):
                e0 = b * _P_B
                for j in range(_P_B // LANES):
                    srcb[p][pl.ds(j * LANES, LANES)] = srcv[
                        pl.ds(e0 + j * LANES, LANES)
                    ]
                    dstb[p][pl.ds(j * LANES, LANES)] = dstv[
                        pl.ds(e0 + j * LANES, LANES)
                    ]

            def gstart(p):
                pltpu.async_copy(tab_hbm.at[srcb[p]], rows[p], semg[p])

            def gwait(p):
                pltpu.make_async_copy(tab_hbm.at[srcb[p]], rows[p],
                                      semg[p]).wait()

            def sstart(p):
                pltpu.async_copy(rows[p], acc.at[dstb[p]], sems[p], add=True)

            def swait(p):
                pltpu.make_async_copy(rows[p], acc.at[dstb[p]],
                                      sems[p]).wait()

            fill(0, 0)
            gstart(0)

            @pl.loop(0, _P_NB // 2)
            def _(i):
                b = 2 * i
                fill(b + 1, 1)
                gstart(1)
                gwait(0)
                sstart(0)
                swait(0)
                fill(b + 2, 0)
                gstart(0)
                gwait(1)
                sstart(1)
                swait(1)

            gwait(0)
            pltpu.sync_copy(rows[0], acc.at[dstb[0]], add=True)

            plsc.subcore_barrier()
            pltpu.sync_copy(
                acc.at[pl.ds(rbase, _P_RPT)], out_hbm.at[pl.ds(rbase, _P_RPT)]
            )

        @pl.when(c == 0)
        def _():
            run(xlo_hbm, olo_hbm)

        @pl.when(c == 1)
        def _():
            run(xhi_hbm, ohi_hbm)

    return k(xs_lo, xs_hi, src, dst)


def _evolve_body(W1_r, Wih1_r, b1_r, W2_r, Wih2_r, b2_r, We1_r, We2_r):
    def evolve(W, Wih, b):
        gates = (
            lax.dot_general(W, Wih, (((1,), (1,)), ((), ())),
                            preferred_element_type=jnp.float32)
            + b
        )
        i = gates[:, :F]
        g = gates[:, 2 * F:3 * F]
        o = gates[:, 3 * F:]
        return jax.nn.sigmoid(o) * jnp.tanh(jax.nn.sigmoid(i) * jnp.tanh(g))

    We1_r[...] = evolve(W1_r[...], Wih1_r[...], b1_r[...])
    We2_r[...] = evolve(W2_r[...], Wih2_r[...], b2_r[...])


@jax.jit
def _tc_evolve(W1, Wih1, b1, W2, Wih2, b2):
    return pl.pallas_call(
        _evolve_body,
        out_shape=[
            jax.ShapeDtypeStruct((F, F), jnp.float32),
            jax.ShapeDtypeStruct((F, F), jnp.float32),
        ],
    )(W1, Wih1, b1, W2, Wih2, b2)


_R = 2000


def _dinv_of(d0_blk, d1_blk):
    return lax.rsqrt(d0_blk + d1_blk + 1.0)


def _mm1_body(x_r, We_r, d0_r, d1_r, lo_r, hi_r):
    xw = jnp.dot(x_r[...], We_r[...], preferred_element_type=jnp.float32)
    xs = _dinv_of(d0_r[...], d1_r[...]) * xw
    lo_r[...] = xs[:, :H]
    hi_r[...] = xs[:, H:]


@jax.jit
def _tc_mm1(x, We1, d0, d1):
    grid = (N // _R,)
    return pl.pallas_call(
        _mm1_body,
        grid=grid,
        in_specs=[
            pl.BlockSpec((_R, F), lambda i: (i, 0)),
            pl.BlockSpec((F, F), lambda i: (0, 0)),
            pl.BlockSpec((_R, 1), lambda i: (i, 0)),
            pl.BlockSpec((_R, 1), lambda i: (i, 0)),
        ],
        out_specs=[
            pl.BlockSpec((_R, H), lambda i: (i, 0)),
            pl.BlockSpec((_R, H), lambda i: (i, 0)),
        ],
        out_shape=[
            jax.ShapeDtypeStruct((N, H), jnp.float32),
            jax.ShapeDtypeStruct((N, H), jnp.float32),
        ],
    )(x, We1, d0, d1)


def _mid_body(alo_r, ahi_r, xlo_r, xhi_r, d0_r, d1_r, We_r, lo_r, hi_r):
    dinv = _dinv_of(d0_r[...], d1_r[...])
    h_lo = jnp.maximum(dinv * (alo_r[...] + xlo_r[...]), 0.0)
    h_hi = jnp.maximum(dinv * (ahi_r[...] + xhi_r[...]), 0.0)
    h = jnp.concatenate([h_lo, h_hi], axis=1)
    ys = dinv * jnp.dot(h, We_r[...], preferred_element_type=jnp.float32)
    lo_r[...] = ys[:, :H]
    hi_r[...] = ys[:, H:]


@jax.jit
def _tc_mid(alo, ahi, xlo, xhi, d0, d1, We2):
    grid = (N // _R,)
    bspec_h = pl.BlockSpec((_R, H), lambda i: (i, 0))
    return pl.pallas_call(
        _mid_body,
        grid=grid,
        in_specs=[
            bspec_h, bspec_h, bspec_h, bspec_h,
            pl.BlockSpec((_R, 1), lambda i: (i, 0)),
            pl.BlockSpec((_R, 1), lambda i: (i, 0)),
            pl.BlockSpec((F, F), lambda i: (0, 0)),
        ],
        out_specs=[bspec_h, bspec_h],
        out_shape=[
            jax.ShapeDtypeStruct((N, H), jnp.float32),
            jax.ShapeDtypeStruct((N, H), jnp.float32),
        ],
    )(alo, ahi, xlo, xhi, d0, d1, We2)


def _final_body(alo_r, ahi_r, xlo_r, xhi_r, d0_r, d1_r, Wlin_r, blin_r,
                out_r):
    dinv = _dinv_of(d0_r[...], d1_r[...])
    h_lo = jnp.maximum(dinv * (alo_r[...] + xlo_r[...]), 0.0)
    h_hi = jnp.maximum(dinv * (ahi_r[...] + xhi_r[...]), 0.0)
    h = jnp.concatenate([h_lo, h_hi], axis=1)
    logits = (
        lax.dot_general(h, Wlin_r[...], (((1,), (1,)), ((), ())),
                        preferred_element_type=jnp.float32)
        + blin_r[...]
    )
    z = logits - jnp.max(logits, axis=-1, keepdims=True)
    lse = jnp.log(jnp.sum(jnp.exp(z), axis=-1, keepdims=True))
    out_r[...] = z - lse


@jax.jit
def _tc_final(alo, ahi, xlo, xhi, d0, d1, Wlin, blin):
    grid = (N // _R,)
    bspec_h = pl.BlockSpec((_R, H), lambda i: (i, 0))
    return pl.pallas_call(
        _final_body,
        grid=grid,
        in_specs=[
            bspec_h, bspec_h, bspec_h, bspec_h,
            pl.BlockSpec((_R, 1), lambda i: (i, 0)),
            pl.BlockSpec((_R, 1), lambda i: (i, 0)),
            pl.BlockSpec((C, F), lambda i: (0, 0)),
            pl.BlockSpec((1, C), lambda i: (0, 0)),
        ],
        out_specs=pl.BlockSpec((_R, C), lambda i: (i, 0)),
        out_shape=jax.ShapeDtypeStruct((N, C), jnp.float32),
    )(alo, ahi, xlo, xhi, d0, d1, Wlin, blin.reshape(1, C))


def kernel(x, edge_index, W1, Wih1, bih1, bhh1, W2, Wih2, bih2, bhh2,
           Wlin, blin):
    src = edge_index[0]
    dst = edge_index[1]

    degv = _sc_degree(dst)
    d0 = degv[:NPAD].reshape(NPAD, 1)
    d1 = degv[NPAD:].reshape(NPAD, 1)

    We1, We2 = _tc_evolve(
        W1, Wih1, (bih1 + bhh1).reshape(1, 4 * F),
        W2, Wih2, (bih2 + bhh2).reshape(1, 4 * F),
    )

    xs_lo, xs_hi = _tc_mm1(x, We1, d0, d1)
    a1_lo, a1_hi = _sc_propagate(xs_lo, xs_hi, src, dst)
    ys_lo, ys_hi = _tc_mid(a1_lo, a1_hi, xs_lo, xs_hi, d0, d1, We2)
    a2_lo, a2_hi = _sc_propagate(ys_lo, ys_hi, src, dst)
    return _tc_final(a2_lo, a2_hi, ys_lo, ys_hi, d0, d1, Wlin, blin)


# speedup vs baseline: 1.0149x; 1.0149x over previous
import dataclasses
import functools

import jax
import jax.numpy as jnp
from jax import lax
from jax.experimental import pallas as pl
from jax.experimental.pallas import tpu as pltpu
from jax.experimental.pallas import tpu_sc as plsc

N = 10000
E = 160000
F = 256
H = 128
C = 16

NC = 2
NS = 16
LANES = 16

NPAD = 10240
_vmesh = plsc.VectorSubcoreMesh(core_axis_name="c", subcore_axis_name="s")


_DEG_EPT = E // (NC * NS)
_DEG_NCH = _DEG_EPT // LANES

_deg_cp = pltpu.CompilerParams()
if "needs_layout_passes" in pltpu.CompilerParams.__dataclass_fields__:
    _deg_cp = dataclasses.replace(_deg_cp, needs_layout_passes=False)


@jax.jit
def _sc_degree(dst):
    @functools.partial(
        pl.kernel,
        mesh=_vmesh,
        compiler_params=_deg_cp,
        out_type=jax.ShapeDtypeStruct((NC * NS, NPAD), jnp.float32),
        scratch_types=[
            pltpu.VMEM((NPAD,), jnp.float32),
            pltpu.VMEM((_DEG_EPT + 16,), jnp.int32),
            pltpu.SemaphoreType.DMA,
        ],
    )
    def k(dst_hbm, out_hbm, hist, dstv, sem):
        c = lax.axis_index("c")
        s = lax.axis_index("s")
        wid = c * NS + s

        cp = pltpu.async_copy(
            dst_hbm.at[pl.ds(wid * _DEG_EPT, _DEG_EPT)],
            dstv.at[pl.ds(0, _DEG_EPT)], sem)

        zero16 = jnp.zeros((LANES,), jnp.float32)

        @pl.loop(0, NPAD // LANES)
        def _(j):
            hist[pl.ds(j * LANES, LANES)] = zero16

        cp.wait()

        ones = jnp.full((LANES,), 1.0, jnp.float32)

        @pl.loop(0, _DEG_NCH)
        def _(j):
            idx = dstv[pl.ds(j * LANES, LANES)]
            plsc.addupdate_scatter(hist, [idx], ones)

        rem_idx = dstv[pl.ds(_DEG_NCH * LANES, LANES)]
        mask = lax.iota(jnp.int32, LANES) < (_DEG_EPT - _DEG_NCH * LANES)
        plsc.addupdate_scatter(hist, [rem_idx], ones, mask=mask)

        pltpu.sync_copy(hist, out_hbm.at[wid])

    return k(dst)


_P_EPT = E // NS
_P_B = 80
_P_NB = _P_EPT // _P_B
_P_RPT = NPAD // NS


@jax.jit
def _sc_propagate(xs_lo, xs_hi, src, dst):
    @functools.partial(
        pl.kernel,
        mesh=_vmesh,
        out_type=[
            jax.ShapeDtypeStruct((NPAD, H), jnp.float32),
            jax.ShapeDtypeStruct((NPAD, H), jnp.float32),
        ],
        scratch_types=[
            pltpu.VMEM_SHARED((NPAD, H), jnp.float32),
            pltpu.VMEM((_P_EPT,), jnp.int32),
            pltpu.VMEM((_P_EPT,), jnp.int32),
        ] + [pltpu.VMEM((_P_B,), jnp.int32) for _ in range(4)] + [
            pltpu.VMEM((_P_B, H), jnp.float32) for _ in range(2)
        ] + [pltpu.SemaphoreType.DMA for _ in range(6)],
    )
    def k(xlo_hbm, xhi_hbm, src_hbm, dst_hbm, olo_hbm, ohi_hbm,
          acc, srcv, dstv,
          sb0, db0, sb1, db1, rw0, rw1,
          semi, semz, sg0, sg1, ss0, ss1):
        c = lax.axis_index("c")
        s = lax.axis_index("s")
        ebase = s * _P_EPT
        rbase = s * _P_RPT
        srcb = [sb0, sb1]
        dstb = [db0, db1]
        rows = [rw0, rw1]
        semg = [sg0, sg1]
        sems = [ss0, ss1]

        def run(tab_hbm, out_hbm):
            cp_s = pltpu.async_copy(src_hbm.at[pl.ds(ebase, _P_EPT)], srcv, semi)
            cp_d = pltpu.async_copy(dst_hbm.at[pl.ds(ebase, _P_EPT)], dstv, semi)

            @pl.loop(0, _P_B)
            def _(r):
                for j in range(H // LANES):
                    rows[0][r, pl.ds(j * LANES, LANES)] = jnp.zeros(
                        (LANES,), jnp.float32
                    )

            zcps = [
                pltpu.async_copy(
                    rows[0], acc.at[pl.ds(rbase + i * _P_B, _P_B)], semz)
                for i in range(_P_RPT // _P_B)
            ]
            for cp in zcps:
                cp.wait()

            cp_s.wait()
            cp_d.wait()
            plsc.subcore_barrier()

            def fill(b, p):
                e0 = b * _P_B
                for j in range(_P_B // LANES):
                    srcb[p][pl.ds(j * LANES, LANES)] = srcv[
                        pl.ds(e0 + j * LANES, LANES)
                    ]
                    dstb[p][pl.ds(j * LANES, LANES)] = dstv[
                        pl.ds(e0 + j * LANES, LANES)
                    ]

            def gstart(p):
                pltpu.async_copy(tab_hbm.at[srcb[p]], rows[p], semg[p])

            def gwait(p):
                pltpu.make_async_copy(tab_hbm.at[srcb[p]], rows[p],
                                      semg[p]).wait()

            def sstart(p):
                pltpu.async_copy(rows[p], acc.at[dstb[p]], sems[p], add=True)

            def swait(p):
                pltpu.make_async_copy(rows[p], acc.at[dstb[p]],
                                      sems[p]).wait()

            fill(0, 0)
            gstart(0)

            @pl.loop(0, _P_NB // 2)
            def _(i):
                b = 2 * i
                fill(b + 1, 1)
                gstart(1)
                gwait(0)
                sstart(0)
                swait(0)
                fill(b + 2, 0)
                gstart(0)
                gwait(1)
                sstart(1)
                swait(1)

            gwait(0)
            pltpu.sync_copy(rows[0], acc.at[dstb[0]], add=True)

            plsc.subcore_barrier()
            pltpu.sync_copy(
                acc.at[pl.ds(rbase, _P_RPT)], out_hbm.at[pl.ds(rbase, _P_RPT)]
            )

        @pl.when(c == 0)
        def _():
            run(xlo_hbm, olo_hbm)

        @pl.when(c == 1)
        def _():
            run(xhi_hbm, ohi_hbm)

    return k(xs_lo, xs_hi, src, dst)


def _evolve_body(W1_r, Wih1_r, b1_r, W2_r, Wih2_r, b2_r, dm_r,
                 We1_r, We2_r, dinv_r):
    def evolve(W, Wih, b):
        gates = (
            lax.dot_general(W, Wih, (((1,), (1,)), ((), ())),
                            preferred_element_type=jnp.float32)
            + b
        )
        i = gates[:, :F]
        g = gates[:, 2 * F:3 * F]
        o = gates[:, 3 * F:]
        return jax.nn.sigmoid(o) * jnp.tanh(jax.nn.sigmoid(i) * jnp.tanh(g))

    We1_r[...] = evolve(W1_r[...], Wih1_r[...], b1_r[...])
    We2_r[...] = evolve(W2_r[...], Wih2_r[...], b2_r[...])
    ones = jnp.full((NC * NS, 1), 1.0, jnp.float32)
    deg = lax.dot_general(dm_r[...], ones, (((0,), (0,)), ((), ())),
                          preferred_element_type=jnp.float32) + 1.0
    dinv_r[...] = lax.rsqrt(deg)


@jax.jit
def _tc_evolve(W1, Wih1, b1, W2, Wih2, b2, degm):
    return pl.pallas_call(
        _evolve_body,
        out_shape=[
            jax.ShapeDtypeStruct((F, F), jnp.float32),
            jax.ShapeDtypeStruct((F, F), jnp.float32),
            jax.ShapeDtypeStruct((NPAD, 1), jnp.float32),
        ],
    )(W1, Wih1, b1, W2, Wih2, b2, degm)


_R = 2000


def _mm1_body(x_r, We_r, dinv_r, lo_r, hi_r):
    xw = jnp.dot(x_r[...], We_r[...], preferred_element_type=jnp.float32)
    xs = dinv_r[...] * xw
    lo_r[...] = xs[:, :H]
    hi_r[...] = xs[:, H:]


@jax.jit
def _tc_mm1(x, We1, dinv):
    grid = (N // _R,)
    return pl.pallas_call(
        _mm1_body,
        grid=grid,
        in_specs=[
            pl.BlockSpec((_R, F), lambda i: (i, 0)),
            pl.BlockSpec((F, F), lambda i: (0, 0)),
            pl.BlockSpec((_R, 1), lambda i: (i, 0)),
        ],
        out_specs=[
            pl.BlockSpec((_R, H), lambda i: (i, 0)),
            pl.BlockSpec((_R, H), lambda i: (i, 0)),
        ],
        out_shape=[
            jax.ShapeDtypeStruct((N, H), jnp.float32),
            jax.ShapeDtypeStruct((N, H), jnp.float32),
        ],
    )(x, We1, dinv)


def _mid_body(alo_r, ahi_r, xlo_r, xhi_r, dinv_r, We_r, lo_r, hi_r):
    dinv = dinv_r[...]
    h_lo = jnp.maximum(dinv * (alo_r[...] + xlo_r[...]), 0.0)
    h_hi = jnp.maximum(dinv * (ahi_r[...] + xhi_r[...]), 0.0)
    h = jnp.concatenate([h_lo, h_hi], axis=1)
    ys = dinv * jnp.dot(h, We_r[...], preferred_element_type=jnp.float32)
    lo_r[...] = ys[:, :H]
    hi_r[...] = ys[:, H:]


@jax.jit
def _tc_mid(alo, ahi, xlo, xhi, dinv, We2):
    grid = (N // _R,)
    bspec_h = pl.BlockSpec((_R, H), lambda i: (i, 0))
    return pl.pallas_call(
        _mid_body,
        grid=grid,
        in_specs=[
            bspec_h, bspec_h, bspec_h, bspec_h,
            pl.BlockSpec((_R, 1), lambda i: (i, 0)),
            pl.BlockSpec((F, F), lambda i: (0, 0)),
        ],
        out_specs=[bspec_h, bspec_h],
        out_shape=[
            jax.ShapeDtypeStruct((N, H), jnp.float32),
            jax.ShapeDtypeStruct((N, H), jnp.float32),
        ],
    )(alo, ahi, xlo, xhi, dinv, We2)


def _final_body(alo_r, ahi_r, xlo_r, xhi_r, dinv_r, Wlin_r, blin_r, out_r):
    dinv = dinv_r[...]
    h_lo = jnp.maximum(dinv * (alo_r[...] + xlo_r[...]), 0.0)
    h_hi = jnp.maximum(dinv * (ahi_r[...] + xhi_r[...]), 0.0)
    h = jnp.concatenate([h_lo, h_hi], axis=1)
    logits = (
        lax.dot_general(h, Wlin_r[...], (((1,), (1,)), ((), ())),
                        preferred_element_type=jnp.float32)
        + blin_r[...]
    )
    z = logits - jnp.max(logits, axis=-1, keepdims=True)
    lse = jnp.log(jnp.sum(jnp.exp(z), axis=-1, keepdims=True))
    out_r[...] = z - lse


@jax.jit
def _tc_final(alo, ahi, xlo, xhi, dinv, Wlin, blin):
    grid = (N // _R,)
    bspec_h = pl.BlockSpec((_R, H), lambda i: (i, 0))
    return pl.pallas_call(
        _final_body,
        grid=grid,
        in_specs=[
            bspec_h, bspec_h, bspec_h, bspec_h,
            pl.BlockSpec((_R, 1), lambda i: (i, 0)),
            pl.BlockSpec((C, F), lambda i: (0, 0)),
            pl.BlockSpec((1, C), lambda i: (0, 0)),
        ],
        out_specs=pl.BlockSpec((_R, C), lambda i: (i, 0)),
        out_shape=jax.ShapeDtypeStruct((N, C), jnp.float32),
    )(alo, ahi, xlo, xhi, dinv, Wlin, blin.reshape(1, C))


def kernel(x, edge_index, W1, Wih1, bih1, bhh1, W2, Wih2, bih2, bhh2,
           Wlin, blin):
    src = edge_index[0]
    dst = edge_index[1]

    degm = _sc_degree(dst)

    We1, We2, dinv = _tc_evolve(
        W1, Wih1, (bih1 + bhh1).reshape(1, 4 * F),
        W2, Wih2, (bih2 + bhh2).reshape(1, 4 * F), degm,
    )

    xs_lo, xs_hi = _tc_mm1(x, We1, dinv)
    a1_lo, a1_hi = _sc_propagate(xs_lo, xs_hi, src, dst)
    ys_lo, ys_hi = _tc_mid(a1_lo, a1_hi, xs_lo, xs_hi, dinv, We2)
    a2_lo, a2_hi = _sc_propagate(ys_lo, ys_hi, src, dst)
    return _tc_final(a2_lo, a2_hi, ys_lo, ys_hi, dinv, Wlin, blin)
